# baseline (speedup 1.0000x reference)
import jax
import jax.numpy as jnp
from jax import lax
from jax.experimental import pallas as pl
from jax.experimental.pallas import tpu as pltpu
from jax.experimental.pallas import tpu_sc as plsc

ROWS = 256
PER_ROW = 3 * 224 * 224
LANE = 128
SUB = PER_ROW // LANE
CHUNKS = ROWS // 16

GRP = 16
NSLOT = 2


def _gather16(vec, idx):
    return lax.gather(
        vec,
        idx[:, None],
        lax.GatherDimensionNumbers(
            offset_dims=(),
            collapsed_slice_dims=(0,),
            start_index_map=(0,),
        ),
        slice_sizes=(1,),
        mode=lax.GatherScatterMode.PROMISE_IN_BOUNDS,
    )


def _compact_body(y_hbm, perm_hbm, n_hbm, y_v, perm_v, n_v):
    cid = lax.axis_index("c")
    sid = lax.axis_index("s")

    @pl.when(jnp.logical_and(cid == 0, sid == 0))
    def _():
        pltpu.sync_copy(y_hbm, y_v)
        lane = lax.iota(jnp.int32, 16)
        last = jnp.full((16,), 15, jnp.int32)
        zero = jnp.zeros((16,), jnp.int32)
        one = jnp.full((16,), 1, jnp.int32)
        for i in range(CHUNKS):
            perm_v[pl.ds(i * 16, 16)] = zero
        base = zero
        for i in range(CHUNKS):
            yv = y_v[pl.ds(i * 16, 16)]
            m = yv == one
            mi = jnp.where(m, one, zero)
            c = plsc.cumsum(mi)
            pos = base + c - mi
            plsc.store_scatter(perm_v, [pos], lane + (i * 16), mask=m)
            base = base + _gather16(c, last)
        n_v[...] = base
        pltpu.sync_copy(perm_v, perm_hbm)
        pltpu.sync_copy(n_v, n_hbm)


_COMPACT_CACHE = []


def _compact(y):
    if not _COMPACT_CACHE:
        _COMPACT_CACHE.append(
            pl.kernel(
                _compact_body,
                out_type=(
                    jax.ShapeDtypeStruct((ROWS,), jnp.int32),
                    jax.ShapeDtypeStruct((16,), jnp.int32),
                ),
                mesh=plsc.VectorSubcoreMesh(
                    core_axis_name="c", subcore_axis_name="s"
                ),
                compiler_params=pltpu.CompilerParams(needs_layout_passes=False),
                scratch_types=[
                    pltpu.VMEM((ROWS,), jnp.int32),
                    pltpu.VMEM((ROWS,), jnp.int32),
                    pltpu.VMEM((16,), jnp.int32),
                ],
            )
        )
    return _COMPACT_CACHE[0](y)


def _loss_body(perm_ref, n_ref, r_hbm, x_hbm, out_ref, rbuf, xbuf, acc, sems):
    n = n_ref[0]
    ngrp = (n + GRP - 1) // GRP

    def row_of(k):
        return perm_ref[lax.min(k, lax.max(n - 1, jnp.int32(0)))]

    def start(g):
        slot = lax.rem(g, NSLOT)
        for j in range(GRP):
            row = row_of(g * GRP + j)
            pltpu.make_async_copy(
                r_hbm.at[row], rbuf.at[slot, j], sems.at[0, slot, j]
            ).start()
            pltpu.make_async_copy(
                x_hbm.at[row], xbuf.at[slot, j], sems.at[1, slot, j]
            ).start()

    for s in range(NSLOT):
        @pl.when(s < ngrp)
        def _():
            start(jnp.int32(s))

    acc[...] = jnp.zeros((8, LANE), jnp.float32)

    def step(g, carry):
        slot = lax.rem(g, NSLOT)
        for j in range(GRP):
            row = row_of(g * GRP + j)
            pltpu.make_async_copy(
                r_hbm.at[row], rbuf.at[slot, j], sems.at[0, slot, j]
            ).wait()
            pltpu.make_async_copy(
                x_hbm.at[row], xbuf.at[slot, j], sems.at[1, slot, j]
            ).wait()
        part = jnp.zeros((8, LANE), jnp.float32)
        for j in range(GRP):
            w = (g * GRP + j < n).astype(jnp.float32)
            d = jnp.abs(rbuf[slot, j] - xbuf[slot, j])
            part = part + w * jnp.sum(d.reshape(SUB // 8, 8, LANE), axis=0)
        acc[...] += part

        @pl.when(g + NSLOT < ngrp)
        def _():
            start(g + NSLOT)

        return carry

    lax.fori_loop(0, ngrp, step, 0)
    total = jnp.sum(acc[...])
    denom = n.astype(jnp.float32) * jnp.float32(PER_ROW)
    out_ref[0, 0] = jnp.where(n > 0, total / denom, jnp.float32(0.0))


_loss = pl.pallas_call(
    _loss_body,
    grid_spec=pltpu.PrefetchScalarGridSpec(
        num_scalar_prefetch=2,
        grid=(1,),
        in_specs=[
            pl.BlockSpec(memory_space=pl.ANY),
            pl.BlockSpec(memory_space=pl.ANY),
        ],
        out_specs=pl.BlockSpec(memory_space=pltpu.SMEM),
        scratch_shapes=[
            pltpu.VMEM((NSLOT, GRP, SUB, LANE), jnp.float32),
            pltpu.VMEM((NSLOT, GRP, SUB, LANE), jnp.float32),
            pltpu.VMEM((8, LANE), jnp.float32),
            pltpu.SemaphoreType.DMA((2, NSLOT, GRP)),
        ],
    ),
    out_shape=jax.ShapeDtypeStruct((1, 1), jnp.float32),
)


def kernel(recons, x, y):
    perm, nvec = _compact(y)
    r3 = recons.reshape(ROWS, SUB, LANE)
    x3 = x.reshape(ROWS, SUB, LANE)
    out = _loss(perm, nvec, r3, x3)
    return out[0, 0]


# speedup vs baseline: 1.0221x; 1.0221x over previous
import jax
import jax.numpy as jnp
from jax import lax
from jax.experimental import pallas as pl
from jax.experimental.pallas import tpu as pltpu
from jax.experimental.pallas import tpu_sc as plsc

ROWS = 256
PER_ROW = 3 * 224 * 224
LANE = 128
SUB = PER_ROW // LANE
CHUNKS = ROWS // 16

GRP = 8
NSLOT = 2


def _gather16(vec, idx):
    return lax.gather(
        vec,
        idx[:, None],
        lax.GatherDimensionNumbers(
            offset_dims=(),
            collapsed_slice_dims=(0,),
            start_index_map=(0,),
        ),
        slice_sizes=(1,),
        mode=lax.GatherScatterMode.PROMISE_IN_BOUNDS,
    )


def _compact_body(y_hbm, perm_hbm, n_hbm, y_v, perm_v, n_v):
    cid = lax.axis_index("c")
    sid = lax.axis_index("s")

    @pl.when(jnp.logical_and(cid == 0, sid == 0))
    def _():
        pltpu.sync_copy(y_hbm, y_v)
        lane = lax.iota(jnp.int32, 16)
        last = jnp.full((16,), 15, jnp.int32)
        zero = jnp.zeros((16,), jnp.int32)
        one = jnp.full((16,), 1, jnp.int32)
        for i in range(CHUNKS):
            perm_v[pl.ds(i * 16, 16)] = zero
        base = zero
        for i in range(CHUNKS):
            yv = y_v[pl.ds(i * 16, 16)]
            m = yv == one
            mi = jnp.where(m, one, zero)
            c = plsc.cumsum(mi)
            pos = base + c - mi
            plsc.store_scatter(perm_v, [pos], lane + (i * 16), mask=m)
            base = base + _gather16(c, last)
        n_v[...] = base
        pltpu.sync_copy(perm_v, perm_hbm)
        pltpu.sync_copy(n_v, n_hbm)


_COMPACT_CACHE = []


def _compact(y):
    if not _COMPACT_CACHE:
        _COMPACT_CACHE.append(
            pl.kernel(
                _compact_body,
                out_type=(
                    jax.ShapeDtypeStruct((ROWS,), jnp.int32),
                    jax.ShapeDtypeStruct((16,), jnp.int32),
                ),
                mesh=plsc.VectorSubcoreMesh(
                    core_axis_name="c", subcore_axis_name="s"
                ),
                compiler_params=pltpu.CompilerParams(needs_layout_passes=False),
                scratch_types=[
                    pltpu.VMEM((ROWS,), jnp.int32),
                    pltpu.VMEM((ROWS,), jnp.int32),
                    pltpu.VMEM((16,), jnp.int32),
                ],
            )
        )
    return _COMPACT_CACHE[0](y)


def _loss_body(perm_ref, n_ref, r_hbm, x_hbm, out_ref, rbuf, xbuf, acc, sems):
    n = n_ref[0]
    ngrp = (n + GRP - 1) // GRP

    def row_of(k):
        return perm_ref[lax.min(k, lax.max(n - 1, jnp.int32(0)))]

    def start(g):
        slot = lax.rem(g, NSLOT)
        for j in range(GRP):
            row = row_of(g * GRP + j)
            pltpu.make_async_copy(
                r_hbm.at[row], rbuf.at[slot, j], sems.at[0, slot, j]
            ).start()
            pltpu.make_async_copy(
                x_hbm.at[row], xbuf.at[slot, j], sems.at[1, slot, j]
            ).start()

    for s in range(NSLOT):
        @pl.when(s < ngrp)
        def _():
            start(jnp.int32(s))

    acc[...] = jnp.zeros((8, LANE), jnp.float32)

    def step(g, carry):
        slot = lax.rem(g, NSLOT)
        for j in range(GRP):
            row = row_of(g * GRP + j)
            pltpu.make_async_copy(
                r_hbm.at[row], rbuf.at[slot, j], sems.at[0, slot, j]
            ).wait()
            pltpu.make_async_copy(
                x_hbm.at[row], xbuf.at[slot, j], sems.at[1, slot, j]
            ).wait()
        part = jnp.zeros((8, LANE), jnp.float32)
        for j in range(GRP):
            w = (g * GRP + j < n).astype(jnp.float32)
            d = jnp.abs(rbuf[slot, j] - xbuf[slot, j])
            part = part + w * jnp.sum(d.reshape(SUB // 8, 8, LANE), axis=0)
        acc[...] += part

        @pl.when(g + NSLOT < ngrp)
        def _():
            start(g + NSLOT)

        return carry

    lax.fori_loop(0, ngrp, step, 0)
    total = jnp.sum(acc[...])
    denom = n.astype(jnp.float32) * jnp.float32(PER_ROW)
    out_ref[0, 0] = jnp.where(n > 0, total / denom, jnp.float32(0.0))


_loss = pl.pallas_call(
    _loss_body,
    grid_spec=pltpu.PrefetchScalarGridSpec(
        num_scalar_prefetch=2,
        grid=(1,),
        in_specs=[
            pl.BlockSpec(memory_space=pl.ANY),
            pl.BlockSpec(memory_space=pl.ANY),
        ],
        out_specs=pl.BlockSpec(memory_space=pltpu.SMEM),
        scratch_shapes=[
            pltpu.VMEM((NSLOT, GRP, SUB, LANE), jnp.float32),
            pltpu.VMEM((NSLOT, GRP, SUB, LANE), jnp.float32),
            pltpu.VMEM((8, LANE), jnp.float32),
            pltpu.SemaphoreType.DMA((2, NSLOT, GRP)),
        ],
    ),
    out_shape=jax.ShapeDtypeStruct((1, 1), jnp.float32),
)


def kernel(recons, x, y):
    perm, nvec = _compact(y)
    r3 = recons.reshape(ROWS, SUB, LANE)
    x3 = x.reshape(ROWS, SUB, LANE)
    out = _loss(perm, nvec, r3, x3)
    return out[0, 0]
